# Initial kernel scaffold; baseline (speedup 1.0000x reference)
import jax
import jax.numpy as jnp
from jax.experimental import pallas as pl


def kernel(x, edge_index, W1, b1, W2, b2):
    raise NotImplementedError("write your pallas kernel here")


# speedup vs baseline: 9.2345x; 9.2345x over previous
import functools

import jax
import jax.numpy as jnp
from jax import lax
from jax.experimental import pallas as pl
from jax.experimental.pallas import tpu as pltpu
from jax.experimental.pallas import tpu_sc as plsc

NN = 10000
EE = 320000
FI = 128
HH = 256
CC = 64

NC = 2
NS = 16
NW = NC * NS
CH = 128
NCHUNK = 80
EP = NW * NCHUNK * CH
NP = 10240
RPT = NP // NS

_MESH = plsc.VectorSubcoreMesh(core_axis_name="c", subcore_axis_name="s")
_SC_PARAMS = pltpu.CompilerParams(use_tc_tiling_on_sc=False)


def _make_deg_kernel():
    @functools.partial(
        pl.kernel,
        mesh=_MESH,
        compiler_params=_SC_PARAMS,
        out_type=jax.ShapeDtypeStruct((NC, NP, 8), jnp.float32),
        scratch_types=[
            pltpu.VMEM((NCHUNK, CH), jnp.int32),
            pltpu.VMEM((CH, 8), jnp.float32),
            pltpu.VMEM_SHARED((NP, 8), jnp.float32),
        ],
    )
    def deg_kernel(dstq_hbm, ones_hbm, zeros_hbm, out_hbm, dst_v, ones_v, acc):
        c = lax.axis_index("c")
        s = lax.axis_index("s")
        wid = c * NS + s
        pltpu.sync_copy(zeros_hbm, acc.at[pl.ds(s * RPT, RPT)])
        pltpu.sync_copy(dstq_hbm.at[wid], dst_v)
        pltpu.sync_copy(ones_hbm, ones_v)
        plsc.subcore_barrier()

        @pl.loop(0, NCHUNK)
        def _(j):
            pltpu.sync_copy(ones_v, acc.at[dst_v.at[j]], add=True)

        plsc.subcore_barrier()
        pltpu.sync_copy(acc.at[pl.ds(s * RPT, RPT)],
                        out_hbm.at[c, pl.ds(s * RPT, RPT)])

    return deg_kernel


def _make_agg_kernel(dd):
    @functools.partial(
        pl.kernel,
        mesh=_MESH,
        compiler_params=_SC_PARAMS,
        out_type=jax.ShapeDtypeStruct((NC, NP, dd), jnp.float32),
        scratch_types=[
            pltpu.VMEM((NCHUNK, CH), jnp.int32),
            pltpu.VMEM((NCHUNK, CH), jnp.int32),
            pltpu.VMEM((CH, dd), jnp.float32),
            pltpu.VMEM_SHARED((NP, dd), jnp.float32),
        ],
    )
    def agg_kernel(table_hbm, srcq_hbm, dstq_hbm, zeros_hbm, out_hbm,
                   src_v, dst_v, rows_v, acc):
        c = lax.axis_index("c")
        s = lax.axis_index("s")
        wid = c * NS + s
        pltpu.sync_copy(zeros_hbm, acc.at[pl.ds(s * RPT, RPT)])
        pltpu.sync_copy(srcq_hbm.at[wid], src_v)
        pltpu.sync_copy(dstq_hbm.at[wid], dst_v)
        plsc.subcore_barrier()

        @pl.loop(0, NCHUNK)
        def _(j):
            pltpu.sync_copy(table_hbm.at[src_v.at[j]], rows_v)
            pltpu.sync_copy(rows_v, acc.at[dst_v.at[j]], add=True)

        plsc.subcore_barrier()
        pltpu.sync_copy(acc.at[pl.ds(s * RPT, RPT)],
                        out_hbm.at[c, pl.ds(s * RPT, RPT)])

    return agg_kernel


_BLK = 256
_GRID = NP // _BLK


def _dinv_block(degp):
    deg = degp[0, :, 0:1] + degp[1, :, 0:1] + 1.0
    return lax.rsqrt(deg)


def _scale_body(degp_ref, x_ref, xs_ref):
    xs_ref[...] = x_ref[...] * _dinv_block(degp_ref[...])


def _mm_body(aggp_ref, degp_ref, xs_ref, w1_ref, b1_ref, w2_ref, p_ref):
    dinv = _dinv_block(degp_ref[...])
    y = dinv * (aggp_ref[0] + aggp_ref[1] + xs_ref[...])
    h = jnp.dot(y, w1_ref[...], preferred_element_type=jnp.float32,
                precision=lax.Precision.HIGHEST)
    h = jnp.maximum(h + b1_ref[...], 0.0)
    p_ref[...] = jnp.dot(h, w2_ref[...], preferred_element_type=jnp.float32,
                         precision=lax.Precision.HIGHEST)


def _comb_body(aggp_ref, b2_ref, o_ref):
    o_ref[...] = aggp_ref[0] + aggp_ref[1] + b2_ref[...]


def kernel(x, edge_index, W1, b1, W2, b2):
    src = edge_index[0].astype(jnp.int32)
    dst = edge_index[1].astype(jnp.int32)
    npad = EP - EE
    srcq = jnp.concatenate([src, jnp.zeros((npad,), jnp.int32)])
    srcq = srcq.reshape(NW, NCHUNK, CH)
    dstq = jnp.concatenate([dst, jnp.full((npad,), NN, jnp.int32)])
    dstq = dstq.reshape(NW, NCHUNK, CH)
    x_pad = jnp.pad(x, ((0, NP - NN), (0, 0)))
    ones8 = jnp.ones((CH, 8), jnp.float32)
    z8 = jnp.zeros((RPT, 8), jnp.float32)
    z128 = jnp.zeros((RPT, FI), jnp.float32)
    z64 = jnp.zeros((RPT, CC), jnp.float32)
    b1r = b1.reshape(1, HH)
    b2r = b2.reshape(1, CC)

    degp = _make_deg_kernel()(dstq, ones8, z8)

    xs = pl.pallas_call(
        _scale_body,
        grid=(_GRID,),
        in_specs=[
            pl.BlockSpec((2, _BLK, 8), lambda i: (0, i, 0)),
            pl.BlockSpec((_BLK, FI), lambda i: (i, 0)),
        ],
        out_specs=pl.BlockSpec((_BLK, FI), lambda i: (i, 0)),
        out_shape=jax.ShapeDtypeStruct((NP, FI), jnp.float32),
    )(degp, x_pad)

    aggp = _make_agg_kernel(FI)(xs, srcq, dstq, z128)

    p = pl.pallas_call(
        _mm_body,
        grid=(_GRID,),
        in_specs=[
            pl.BlockSpec((2, _BLK, FI), lambda i: (0, i, 0)),
            pl.BlockSpec((2, _BLK, 8), lambda i: (0, i, 0)),
            pl.BlockSpec((_BLK, FI), lambda i: (i, 0)),
            pl.BlockSpec((FI, HH), lambda i: (0, 0)),
            pl.BlockSpec((1, HH), lambda i: (0, 0)),
            pl.BlockSpec((HH, CC), lambda i: (0, 0)),
        ],
        out_specs=pl.BlockSpec((_BLK, CC), lambda i: (i, 0)),
        out_shape=jax.ShapeDtypeStruct((NP, CC), jnp.float32),
    )(aggp, degp, xs, W1, b1r, W2)

    agg2p = _make_agg_kernel(CC)(p, srcq, dstq, z64)

    out = pl.pallas_call(
        _comb_body,
        grid=(_GRID,),
        in_specs=[
            pl.BlockSpec((2, _BLK, CC), lambda i: (0, i, 0)),
            pl.BlockSpec((1, CC), lambda i: (0, 0)),
        ],
        out_specs=pl.BlockSpec((_BLK, CC), lambda i: (i, 0)),
        out_shape=jax.ShapeDtypeStruct((NP, CC), jnp.float32),
    )(agg2p, b2r)

    return out[:NN]


# speedup vs baseline: 10.4333x; 1.1298x over previous
import functools

import jax
import jax.numpy as jnp
from jax import lax
from jax.experimental import pallas as pl
from jax.experimental.pallas import tpu as pltpu
from jax.experimental.pallas import tpu_sc as plsc

NN = 10000
EE = 320000
FI = 128
HH = 256
CC = 64

NC = 2
NS = 16
NW = NC * NS
CH = 128
NCHUNK = 80
EP = NW * NCHUNK * CH
NP = 10240
RPT = NP // NS

_MESH = plsc.VectorSubcoreMesh(core_axis_name="c", subcore_axis_name="s")
_SC_PARAMS = pltpu.CompilerParams(use_tc_tiling_on_sc=False)


def _make_deg_kernel():
    @functools.partial(
        pl.kernel,
        mesh=_MESH,
        compiler_params=_SC_PARAMS,
        out_type=jax.ShapeDtypeStruct((NC, NP, 8), jnp.float32),
        scratch_types=[
            pltpu.VMEM((NCHUNK, CH), jnp.int32),
            pltpu.VMEM((CH, 8), jnp.float32),
            pltpu.VMEM_SHARED((NP, 8), jnp.float32),
        ],
    )
    def deg_kernel(dstq_hbm, ones_hbm, zeros_hbm, out_hbm, dst_v, ones_v, acc):
        c = lax.axis_index("c")
        s = lax.axis_index("s")
        wid = c * NS + s
        pltpu.sync_copy(zeros_hbm, acc.at[pl.ds(s * RPT, RPT)])
        pltpu.sync_copy(dstq_hbm.at[wid], dst_v)
        pltpu.sync_copy(ones_hbm, ones_v)
        plsc.subcore_barrier()

        @pl.loop(0, NCHUNK)
        def _(j):
            pltpu.sync_copy(ones_v, acc.at[dst_v.at[j]], add=True)

        plsc.subcore_barrier()
        pltpu.sync_copy(acc.at[pl.ds(s * RPT, RPT)],
                        out_hbm.at[c, pl.ds(s * RPT, RPT)])

    return deg_kernel


def _make_agg_kernel(dd):
    @functools.partial(
        pl.kernel,
        mesh=_MESH,
        compiler_params=_SC_PARAMS,
        out_type=jax.ShapeDtypeStruct((NC, NP, dd), jnp.float32),
        scratch_types=[
            pltpu.VMEM((NCHUNK, CH), jnp.int32),
            pltpu.VMEM((CH,), jnp.int32),
            pltpu.VMEM((CH,), jnp.int32),
            pltpu.VMEM((CH, dd), jnp.float32),
            pltpu.VMEM((CH, dd), jnp.float32),
            pltpu.VMEM_SHARED((NP, dd), jnp.float32),
            pltpu.SemaphoreType.DMA,
            pltpu.SemaphoreType.DMA,
            pltpu.SemaphoreType.DMA,
            pltpu.SemaphoreType.DMA,
        ],
    )
    def agg_kernel(table_hbm, srcq_hbm, dstq_hbm, zeros_hbm, out_hbm,
                   src_v, dst_a, dst_b, rows_a, rows_b, acc,
                   sem_a, sem_b, sem_da, sem_db):
        c = lax.axis_index("c")
        s = lax.axis_index("s")
        wid = c * NS + s
        pltpu.sync_copy(zeros_hbm, acc.at[pl.ds(s * RPT, RPT)])
        pltpu.sync_copy(srcq_hbm.at[wid], src_v)
        plsc.subcore_barrier()

        pltpu.async_copy(dstq_hbm.at[wid, 0], dst_a, sem_da)
        pltpu.async_copy(dstq_hbm.at[wid, 1], dst_b, sem_db)
        pltpu.async_copy(table_hbm.at[src_v.at[0]], rows_a, sem_a)
        pltpu.async_copy(table_hbm.at[src_v.at[1]], rows_b, sem_b)

        @pl.loop(0, NCHUNK - 2, step=2)
        def _(j):
            pltpu.make_async_copy(dstq_hbm.at[wid, j], dst_a, sem_da).wait()
            pltpu.make_async_copy(table_hbm.at[src_v.at[j]], rows_a,
                                  sem_a).wait()
            pltpu.sync_copy(rows_a, acc.at[dst_a], add=True)
            pltpu.async_copy(dstq_hbm.at[wid, j + 2], dst_a, sem_da)
            pltpu.async_copy(table_hbm.at[src_v.at[j + 2]], rows_a, sem_a)
            pltpu.make_async_copy(dstq_hbm.at[wid, j + 1], dst_b,
                                  sem_db).wait()
            pltpu.make_async_copy(table_hbm.at[src_v.at[j + 1]], rows_b,
                                  sem_b).wait()
            pltpu.sync_copy(rows_b, acc.at[dst_b], add=True)
            pltpu.async_copy(dstq_hbm.at[wid, j + 3], dst_b, sem_db)
            pltpu.async_copy(table_hbm.at[src_v.at[j + 3]], rows_b, sem_b)

        pltpu.make_async_copy(dstq_hbm.at[wid, 0], dst_a, sem_da).wait()
        pltpu.make_async_copy(table_hbm.at[src_v.at[NCHUNK - 2]], rows_a,
                              sem_a).wait()
        pltpu.sync_copy(rows_a, acc.at[dst_a], add=True)
        pltpu.make_async_copy(dstq_hbm.at[wid, 0], dst_b, sem_db).wait()
        pltpu.make_async_copy(table_hbm.at[src_v.at[NCHUNK - 1]], rows_b,
                              sem_b).wait()
        pltpu.sync_copy(rows_b, acc.at[dst_b], add=True)

        plsc.subcore_barrier()
        pltpu.sync_copy(acc.at[pl.ds(s * RPT, RPT)],
                        out_hbm.at[c, pl.ds(s * RPT, RPT)])

    return agg_kernel


_BLK = 256
_GRID = NP // _BLK


def _dinv_block(degp):
    deg = degp[0, :, 0:1] + degp[1, :, 0:1] + 1.0
    return lax.rsqrt(deg)


def _scale_body(degp_ref, x_ref, xs_ref):
    xs_ref[...] = x_ref[...] * _dinv_block(degp_ref[...])


def _mm_body(aggp_ref, degp_ref, xs_ref, w1_ref, b1_ref, w2_ref, p_ref):
    dinv = _dinv_block(degp_ref[...])
    y = dinv * (aggp_ref[0] + aggp_ref[1] + xs_ref[...])
    h = jnp.dot(y, w1_ref[...], preferred_element_type=jnp.float32,
                precision=lax.Precision.HIGHEST)
    h = jnp.maximum(h + b1_ref[...], 0.0)
    p_ref[...] = jnp.dot(h, w2_ref[...], preferred_element_type=jnp.float32,
                         precision=lax.Precision.HIGHEST)


def _comb_body(aggp_ref, b2_ref, o_ref):
    o_ref[...] = aggp_ref[0] + aggp_ref[1] + b2_ref[...]


def kernel(x, edge_index, W1, b1, W2, b2):
    src = edge_index[0].astype(jnp.int32)
    dst = edge_index[1].astype(jnp.int32)
    npad = EP - EE
    srcq = jnp.concatenate([src, jnp.zeros((npad,), jnp.int32)])
    srcq = srcq.reshape(NW, NCHUNK, CH)
    dstq = jnp.concatenate([dst, jnp.full((npad,), NN, jnp.int32)])
    dstq = dstq.reshape(NW, NCHUNK, CH)
    x_pad = jnp.pad(x, ((0, NP - NN), (0, 0)))
    ones8 = jnp.ones((CH, 8), jnp.float32)
    z8 = jnp.zeros((RPT, 8), jnp.float32)
    z128 = jnp.zeros((RPT, FI), jnp.float32)
    z64 = jnp.zeros((RPT, CC), jnp.float32)
    b1r = b1.reshape(1, HH)
    b2r = b2.reshape(1, CC)

    degp = _make_deg_kernel()(dstq, ones8, z8)

    xs = pl.pallas_call(
        _scale_body,
        grid=(_GRID,),
        in_specs=[
            pl.BlockSpec((2, _BLK, 8), lambda i: (0, i, 0)),
            pl.BlockSpec((_BLK, FI), lambda i: (i, 0)),
        ],
        out_specs=pl.BlockSpec((_BLK, FI), lambda i: (i, 0)),
        out_shape=jax.ShapeDtypeStruct((NP, FI), jnp.float32),
    )(degp, x_pad)

    aggp = _make_agg_kernel(FI)(xs, srcq, dstq, z128)

    p = pl.pallas_call(
        _mm_body,
        grid=(_GRID,),
        in_specs=[
            pl.BlockSpec((2, _BLK, FI), lambda i: (0, i, 0)),
            pl.BlockSpec((2, _BLK, 8), lambda i: (0, i, 0)),
            pl.BlockSpec((_BLK, FI), lambda i: (i, 0)),
            pl.BlockSpec((FI, HH), lambda i: (0, 0)),
            pl.BlockSpec((1, HH), lambda i: (0, 0)),
            pl.BlockSpec((HH, CC), lambda i: (0, 0)),
        ],
        out_specs=pl.BlockSpec((_BLK, CC), lambda i: (i, 0)),
        out_shape=jax.ShapeDtypeStruct((NP, CC), jnp.float32),
    )(aggp, degp, xs, W1, b1r, W2)

    agg2p = _make_agg_kernel(CC)(p, srcq, dstq, z64)

    out = pl.pallas_call(
        _comb_body,
        grid=(_GRID,),
        in_specs=[
            pl.BlockSpec((2, _BLK, CC), lambda i: (0, i, 0)),
            pl.BlockSpec((1, CC), lambda i: (0, 0)),
        ],
        out_specs=pl.BlockSpec((_BLK, CC), lambda i: (i, 0)),
        out_shape=jax.ShapeDtypeStruct((NP, CC), jnp.float32),
    )(agg2p, b2r)

    return out[:NN]


# speedup vs baseline: 21.3864x; 2.0498x over previous
import functools

import jax
import jax.numpy as jnp
from jax import lax
from jax.experimental import pallas as pl
from jax.experimental.pallas import tpu as pltpu
from jax.experimental.pallas import tpu_sc as plsc

NN = 10000
EE = 320000
FI = 128
HH = 256
CC = 64

NC = 2
NS = 16
CH = 128
NCH2 = 160
NCHD = NCH2 // NC
EP = NS * NCH2 * CH
NP = 10240
RPT = NP // NS

_MESH = plsc.VectorSubcoreMesh(core_axis_name="c", subcore_axis_name="s")
_SC_PARAMS = pltpu.CompilerParams(use_tc_tiling_on_sc=False)


def _make_deg_kernel():
    @functools.partial(
        pl.kernel,
        mesh=_MESH,
        compiler_params=_SC_PARAMS,
        out_type=jax.ShapeDtypeStruct((NC, NP, 8), jnp.float32),
        scratch_types=[
            pltpu.VMEM((NCHD, CH), jnp.int32),
            pltpu.VMEM((CH, 8), jnp.float32),
            pltpu.VMEM_SHARED((NP, 8), jnp.float32),
        ],
    )
    def deg_kernel(dstq_hbm, ones_hbm, zeros_hbm, out_hbm, dst_v, ones_v, acc):
        c = lax.axis_index("c")
        s = lax.axis_index("s")
        pltpu.sync_copy(zeros_hbm, acc.at[pl.ds(s * RPT, RPT)])
        pltpu.sync_copy(dstq_hbm.at[s, pl.ds(c * NCHD, NCHD)], dst_v)
        pltpu.sync_copy(ones_hbm, ones_v)
        plsc.subcore_barrier()

        @pl.loop(0, NCHD)
        def _(j):
            pltpu.sync_copy(ones_v, acc.at[dst_v.at[j]], add=True)

        plsc.subcore_barrier()
        pltpu.sync_copy(acc.at[pl.ds(s * RPT, RPT)],
                        out_hbm.at[c, pl.ds(s * RPT, RPT)])

    return deg_kernel


def _make_agg_kernel(dh):
    @functools.partial(
        pl.kernel,
        mesh=_MESH,
        compiler_params=_SC_PARAMS,
        out_type=jax.ShapeDtypeStruct((NC, NP, dh), jnp.float32),
        scratch_types=[
            pltpu.VMEM((NCH2, CH), jnp.int32),
            pltpu.VMEM((CH,), jnp.int32),
            pltpu.VMEM((CH,), jnp.int32),
            pltpu.VMEM((CH, dh), jnp.float32),
            pltpu.VMEM((CH, dh), jnp.float32),
            pltpu.VMEM_SHARED((NP, dh), jnp.float32),
            pltpu.VMEM_SHARED((NP, dh), jnp.float32),
            pltpu.SemaphoreType.DMA,
            pltpu.SemaphoreType.DMA,
            pltpu.SemaphoreType.DMA,
            pltpu.SemaphoreType.DMA,
        ],
    )
    def agg_kernel(table2_hbm, srcq_hbm, dstq_hbm, zeros_hbm, out_hbm,
                   src_v, dst_a, dst_b, rows_a, rows_b, table_s, acc,
                   sem_a, sem_b, sem_da, sem_db):
        c = lax.axis_index("c")
        s = lax.axis_index("s")
        pltpu.sync_copy(zeros_hbm, acc.at[pl.ds(s * RPT, RPT)])
        pltpu.sync_copy(table2_hbm.at[c, pl.ds(s * RPT, RPT)],
                        table_s.at[pl.ds(s * RPT, RPT)])
        pltpu.sync_copy(srcq_hbm.at[s], src_v)
        plsc.subcore_barrier()

        pltpu.async_copy(dstq_hbm.at[s, 0], dst_a, sem_da)
        pltpu.async_copy(dstq_hbm.at[s, 1], dst_b, sem_db)
        pltpu.async_copy(table_s.at[src_v.at[0]], rows_a, sem_a)
        pltpu.async_copy(table_s.at[src_v.at[1]], rows_b, sem_b)

        @pl.loop(0, NCH2 - 2, step=2)
        def _(j):
            pltpu.make_async_copy(dstq_hbm.at[s, j], dst_a, sem_da).wait()
            pltpu.make_async_copy(table_s.at[src_v.at[j]], rows_a,
                                  sem_a).wait()
            pltpu.sync_copy(rows_a, acc.at[dst_a], add=True)
            pltpu.async_copy(dstq_hbm.at[s, j + 2], dst_a, sem_da)
            pltpu.async_copy(table_s.at[src_v.at[j + 2]], rows_a, sem_a)
            pltpu.make_async_copy(dstq_hbm.at[s, j + 1], dst_b,
                                  sem_db).wait()
            pltpu.make_async_copy(table_s.at[src_v.at[j + 1]], rows_b,
                                  sem_b).wait()
            pltpu.sync_copy(rows_b, acc.at[dst_b], add=True)
            pltpu.async_copy(dstq_hbm.at[s, j + 3], dst_b, sem_db)
            pltpu.async_copy(table_s.at[src_v.at[j + 3]], rows_b, sem_b)

        pltpu.make_async_copy(dstq_hbm.at[s, 0], dst_a, sem_da).wait()
        pltpu.make_async_copy(table_s.at[src_v.at[NCH2 - 2]], rows_a,
                              sem_a).wait()
        pltpu.sync_copy(rows_a, acc.at[dst_a], add=True)
        pltpu.make_async_copy(dstq_hbm.at[s, 0], dst_b, sem_db).wait()
        pltpu.make_async_copy(table_s.at[src_v.at[NCH2 - 1]], rows_b,
                              sem_b).wait()
        pltpu.sync_copy(rows_b, acc.at[dst_b], add=True)

        plsc.subcore_barrier()
        pltpu.sync_copy(acc.at[pl.ds(s * RPT, RPT)],
                        out_hbm.at[c, pl.ds(s * RPT, RPT)])

    return agg_kernel


_BLK = 256
_GRID = NP // _BLK
FH = FI // 2
CH2 = CC // 2


def _dinv_block(degp):
    deg = degp[0, :, 0:1] + degp[1, :, 0:1] + 1.0
    return lax.rsqrt(deg)


def _scale_body(degp_ref, x_ref, xs2_ref):
    dinv = _dinv_block(degp_ref[...])
    xs2_ref[0] = x_ref[:, :FH] * dinv
    xs2_ref[1] = x_ref[:, FH:] * dinv


def _mm_body(s2_ref, degp_ref, xs2_ref, w1_ref, b1_ref, w2_ref, p2_ref):
    dinv = _dinv_block(degp_ref[...])
    y0 = dinv * (s2_ref[0] + xs2_ref[0])
    y1 = dinv * (s2_ref[1] + xs2_ref[1])
    h = (jnp.dot(y0, w1_ref[0], preferred_element_type=jnp.float32,
                 precision=lax.Precision.HIGHEST)
         + jnp.dot(y1, w1_ref[1], preferred_element_type=jnp.float32,
                   precision=lax.Precision.HIGHEST))
    h = jnp.maximum(h + b1_ref[...], 0.0)
    p = jnp.dot(h, w2_ref[...], preferred_element_type=jnp.float32,
                precision=lax.Precision.HIGHEST)
    p2_ref[0] = p[:, :CH2]
    p2_ref[1] = p[:, CH2:]


def _comb_body(a2_ref, b2_ref, o_ref):
    o_ref[:, :CH2] = a2_ref[0] + b2_ref[:, :CH2]
    o_ref[:, CH2:] = a2_ref[1] + b2_ref[:, CH2:]


def kernel(x, edge_index, W1, b1, W2, b2):
    src = edge_index[0].astype(jnp.int32)
    dst = edge_index[1].astype(jnp.int32)
    npad = EP - EE
    srcq = jnp.concatenate([src, jnp.zeros((npad,), jnp.int32)])
    srcq = srcq.reshape(NS, NCH2, CH)
    dstq = jnp.concatenate([dst, jnp.full((npad,), NN, jnp.int32)])
    dstq = dstq.reshape(NS, NCH2, CH)
    x_pad = jnp.pad(x, ((0, NP - NN), (0, 0)))
    ones8 = jnp.ones((CH, 8), jnp.float32)
    z8 = jnp.zeros((RPT, 8), jnp.float32)
    zfh = jnp.zeros((RPT, FH), jnp.float32)
    zch = jnp.zeros((RPT, CH2), jnp.float32)
    w1s = W1.reshape(2, FH, HH)
    b1r = b1.reshape(1, HH)
    b2r = b2.reshape(1, CC)

    degp = _make_deg_kernel()(dstq, ones8, z8)

    xs2 = pl.pallas_call(
        _scale_body,
        grid=(_GRID,),
        in_specs=[
            pl.BlockSpec((2, _BLK, 8), lambda i: (0, i, 0)),
            pl.BlockSpec((_BLK, FI), lambda i: (i, 0)),
        ],
        out_specs=pl.BlockSpec((2, _BLK, FH), lambda i: (0, i, 0)),
        out_shape=jax.ShapeDtypeStruct((2, NP, FH), jnp.float32),
    )(degp, x_pad)

    s2 = _make_agg_kernel(FH)(xs2, srcq, dstq, zfh)

    p2 = pl.pallas_call(
        _mm_body,
        grid=(_GRID,),
        in_specs=[
            pl.BlockSpec((2, _BLK, FH), lambda i: (0, i, 0)),
            pl.BlockSpec((2, _BLK, 8), lambda i: (0, i, 0)),
            pl.BlockSpec((2, _BLK, FH), lambda i: (0, i, 0)),
            pl.BlockSpec((2, FH, HH), lambda i: (0, 0, 0)),
            pl.BlockSpec((1, HH), lambda i: (0, 0)),
            pl.BlockSpec((HH, CC), lambda i: (0, 0)),
        ],
        out_specs=pl.BlockSpec((2, _BLK, CH2), lambda i: (0, i, 0)),
        out_shape=jax.ShapeDtypeStruct((2, NP, CH2), jnp.float32),
    )(s2, degp, xs2, w1s, b1r, W2)

    a2 = _make_agg_kernel(CH2)(p2, srcq, dstq, zch)

    out = pl.pallas_call(
        _comb_body,
        grid=(_GRID,),
        in_specs=[
            pl.BlockSpec((2, _BLK, CH2), lambda i: (0, i, 0)),
            pl.BlockSpec((1, CC), lambda i: (0, 0)),
        ],
        out_specs=pl.BlockSpec((_BLK, CC), lambda i: (i, 0)),
        out_shape=jax.ShapeDtypeStruct((NP, CC), jnp.float32),
    )(a2, b2r)

    return out[:NN]


# speedup vs baseline: 21.5480x; 1.0076x over previous
import functools

import jax
import jax.numpy as jnp
from jax import lax
from jax.experimental import pallas as pl
from jax.experimental.pallas import tpu as pltpu
from jax.experimental.pallas import tpu_sc as plsc

NN = 10000
EE = 320000
FI = 128
HH = 256
CC = 64

NC = 2
NS = 16
CH = 128
NCH2 = 162
NCHD = NCH2 // NC
EP = NS * NCH2 * CH
NP = 10240
RPT = NP // NS

_MESH = plsc.VectorSubcoreMesh(core_axis_name="c", subcore_axis_name="s")
_SC_PARAMS = pltpu.CompilerParams(use_tc_tiling_on_sc=False)


def _make_deg_kernel():
    @functools.partial(
        pl.kernel,
        mesh=_MESH,
        compiler_params=_SC_PARAMS,
        out_type=jax.ShapeDtypeStruct((NC, NP, 8), jnp.float32),
        scratch_types=[
            pltpu.VMEM((NCHD, CH), jnp.int32),
            pltpu.VMEM((CH, 8), jnp.float32),
            pltpu.VMEM_SHARED((NP, 8), jnp.float32),
        ],
    )
    def deg_kernel(dstq_hbm, ones_hbm, zeros_hbm, out_hbm, dst_v, ones_v, acc):
        c = lax.axis_index("c")
        s = lax.axis_index("s")
        pltpu.sync_copy(zeros_hbm, acc.at[pl.ds(s * RPT, RPT)])
        pltpu.sync_copy(dstq_hbm.at[s, pl.ds(c * NCHD, NCHD)], dst_v)
        pltpu.sync_copy(ones_hbm, ones_v)
        plsc.subcore_barrier()

        @pl.loop(0, NCHD)
        def _(j):
            pltpu.sync_copy(ones_v, acc.at[dst_v.at[j]], add=True)

        plsc.subcore_barrier()
        pltpu.sync_copy(acc.at[pl.ds(s * RPT, RPT)],
                        out_hbm.at[c, pl.ds(s * RPT, RPT)])

    return deg_kernel


def _make_agg_kernel(dh):
    nbuf = 3

    @functools.partial(
        pl.kernel,
        mesh=_MESH,
        compiler_params=_SC_PARAMS,
        out_type=jax.ShapeDtypeStruct((NC, NP, dh), jnp.float32),
        scratch_types=(
            [pltpu.VMEM((NCH2, CH), jnp.int32)]
            + [pltpu.VMEM((CH,), jnp.int32) for _ in range(nbuf)]
            + [pltpu.VMEM((CH, dh), jnp.float32) for _ in range(nbuf)]
            + [pltpu.VMEM_SHARED((NP, dh), jnp.float32)] * 2
            + [pltpu.SemaphoreType.DMA] * (2 * nbuf)
        ),
    )
    def agg_kernel(table2_hbm, srcq_hbm, dstq_hbm, zeros_hbm, out_hbm,
                   *refs):
        src_v = refs[0]
        dst_b = refs[1:1 + nbuf]
        rows_b = refs[1 + nbuf:1 + 2 * nbuf]
        table_s, acc = refs[1 + 2 * nbuf], refs[2 + 2 * nbuf]
        sem_r = refs[3 + 2 * nbuf:3 + 3 * nbuf]
        sem_d = refs[3 + 3 * nbuf:3 + 4 * nbuf]
        c = lax.axis_index("c")
        s = lax.axis_index("s")
        pltpu.sync_copy(zeros_hbm, acc.at[pl.ds(s * RPT, RPT)])
        pltpu.sync_copy(table2_hbm.at[c, pl.ds(s * RPT, RPT)],
                        table_s.at[pl.ds(s * RPT, RPT)])
        pltpu.sync_copy(srcq_hbm.at[s], src_v)
        plsc.subcore_barrier()

        for k in range(nbuf):
            pltpu.async_copy(dstq_hbm.at[s, k], dst_b[k], sem_d[k])
            pltpu.async_copy(table_s.at[src_v.at[k]], rows_b[k], sem_r[k])

        @pl.loop(0, NCH2 - nbuf, step=nbuf)
        def _(j):
            for k in range(nbuf):
                pltpu.make_async_copy(dstq_hbm.at[s, j], dst_b[k],
                                      sem_d[k]).wait()
                pltpu.make_async_copy(table_s.at[src_v.at[j]], rows_b[k],
                                      sem_r[k]).wait()
                pltpu.sync_copy(rows_b[k], acc.at[dst_b[k]], add=True)
                pltpu.async_copy(dstq_hbm.at[s, j + nbuf + k], dst_b[k],
                                 sem_d[k])
                pltpu.async_copy(table_s.at[src_v.at[j + nbuf + k]],
                                 rows_b[k], sem_r[k])

        for k in range(nbuf):
            pltpu.make_async_copy(dstq_hbm.at[s, 0], dst_b[k],
                                  sem_d[k]).wait()
            pltpu.make_async_copy(table_s.at[src_v.at[0]], rows_b[k],
                                  sem_r[k]).wait()
            pltpu.sync_copy(rows_b[k], acc.at[dst_b[k]], add=True)

        plsc.subcore_barrier()
        pltpu.sync_copy(acc.at[pl.ds(s * RPT, RPT)],
                        out_hbm.at[c, pl.ds(s * RPT, RPT)])

    return agg_kernel


_BLK = 256
_GRID = NP // _BLK
FH = FI // 2
CH2 = CC // 2


def _dinv_block(degp):
    deg = degp[0, :, 0:1] + degp[1, :, 0:1] + 1.0
    return lax.rsqrt(deg)


def _scale_body(degp_ref, x_ref, xs2_ref):
    dinv = _dinv_block(degp_ref[...])
    xs2_ref[0] = x_ref[:, :FH] * dinv
    xs2_ref[1] = x_ref[:, FH:] * dinv


def _mm_body(s2_ref, degp_ref, xs2_ref, w1_ref, b1_ref, w2_ref, p2_ref):
    dinv = _dinv_block(degp_ref[...])
    y0 = dinv * (s2_ref[0] + xs2_ref[0])
    y1 = dinv * (s2_ref[1] + xs2_ref[1])
    h = (jnp.dot(y0, w1_ref[0], preferred_element_type=jnp.float32,
                 precision=lax.Precision.HIGHEST)
         + jnp.dot(y1, w1_ref[1], preferred_element_type=jnp.float32,
                   precision=lax.Precision.HIGHEST))
    h = jnp.maximum(h + b1_ref[...], 0.0)
    p = jnp.dot(h, w2_ref[...], preferred_element_type=jnp.float32,
                precision=lax.Precision.HIGHEST)
    p2_ref[0] = p[:, :CH2]
    p2_ref[1] = p[:, CH2:]


def _comb_body(a2_ref, b2_ref, o_ref):
    o_ref[:, :CH2] = a2_ref[0] + b2_ref[:, :CH2]
    o_ref[:, CH2:] = a2_ref[1] + b2_ref[:, CH2:]


def kernel(x, edge_index, W1, b1, W2, b2):
    src = edge_index[0].astype(jnp.int32)
    dst = edge_index[1].astype(jnp.int32)
    npad = EP - EE
    srcq = jnp.concatenate([src, jnp.zeros((npad,), jnp.int32)])
    srcq = srcq.reshape(NS, NCH2, CH)
    dstq = jnp.concatenate([dst, jnp.full((npad,), NN, jnp.int32)])
    dstq = dstq.reshape(NS, NCH2, CH)
    x_pad = jnp.pad(x, ((0, NP - NN), (0, 0)))
    ones8 = jnp.ones((CH, 8), jnp.float32)
    z8 = jnp.zeros((RPT, 8), jnp.float32)
    zfh = jnp.zeros((RPT, FH), jnp.float32)
    zch = jnp.zeros((RPT, CH2), jnp.float32)
    w1s = W1.reshape(2, FH, HH)
    b1r = b1.reshape(1, HH)
    b2r = b2.reshape(1, CC)

    degp = _make_deg_kernel()(dstq, ones8, z8)

    xs2 = pl.pallas_call(
        _scale_body,
        grid=(_GRID,),
        in_specs=[
            pl.BlockSpec((2, _BLK, 8), lambda i: (0, i, 0)),
            pl.BlockSpec((_BLK, FI), lambda i: (i, 0)),
        ],
        out_specs=pl.BlockSpec((2, _BLK, FH), lambda i: (0, i, 0)),
        out_shape=jax.ShapeDtypeStruct((2, NP, FH), jnp.float32),
    )(degp, x_pad)

    s2 = _make_agg_kernel(FH)(xs2, srcq, dstq, zfh)

    p2 = pl.pallas_call(
        _mm_body,
        grid=(_GRID,),
        in_specs=[
            pl.BlockSpec((2, _BLK, FH), lambda i: (0, i, 0)),
            pl.BlockSpec((2, _BLK, 8), lambda i: (0, i, 0)),
            pl.BlockSpec((2, _BLK, FH), lambda i: (0, i, 0)),
            pl.BlockSpec((2, FH, HH), lambda i: (0, 0, 0)),
            pl.BlockSpec((1, HH), lambda i: (0, 0)),
            pl.BlockSpec((HH, CC), lambda i: (0, 0)),
        ],
        out_specs=pl.BlockSpec((2, _BLK, CH2), lambda i: (0, i, 0)),
        out_shape=jax.ShapeDtypeStruct((2, NP, CH2), jnp.float32),
    )(s2, degp, xs2, w1s, b1r, W2)

    a2 = _make_agg_kernel(CH2)(p2, srcq, dstq, zch)

    out = pl.pallas_call(
        _comb_body,
        grid=(_GRID,),
        in_specs=[
            pl.BlockSpec((2, _BLK, CH2), lambda i: (0, i, 0)),
            pl.BlockSpec((1, CC), lambda i: (0, 0)),
        ],
        out_specs=pl.BlockSpec((_BLK, CC), lambda i: (i, 0)),
        out_shape=jax.ShapeDtypeStruct((NP, CC), jnp.float32),
    )(a2, b2r)

    return out[:NN]


# speedup vs baseline: 23.1582x; 1.0747x over previous
import functools

import jax
import jax.numpy as jnp
from jax import lax
from jax.experimental import pallas as pl
from jax.experimental.pallas import tpu as pltpu
from jax.experimental.pallas import tpu_sc as plsc

NN = 10000
EE = 320000
FI = 128
HH = 256
CC = 64

NC = 2
NS = 16
CH = 128
NCH2 = 162
NCHD = NCH2 // NC
EP = NS * NCH2 * CH
NP = 10240
RPT = NP // NS

_MESH = plsc.VectorSubcoreMesh(core_axis_name="c", subcore_axis_name="s")
_SC_PARAMS = pltpu.CompilerParams(use_tc_tiling_on_sc=False)


def _make_deg_kernel():
    @functools.partial(
        pl.kernel,
        mesh=_MESH,
        compiler_params=_SC_PARAMS,
        out_type=jax.ShapeDtypeStruct((NC, NP, 8), jnp.float32),
        scratch_types=[
            pltpu.VMEM((NCHD, CH), jnp.int32),
            pltpu.VMEM((CH, 8), jnp.float32),
            pltpu.VMEM_SHARED((NP, 8), jnp.float32),
        ],
    )
    def deg_kernel(dstq_hbm, ones_hbm, zeros_hbm, out_hbm, dst_v, ones_v, acc):
        c = lax.axis_index("c")
        s = lax.axis_index("s")
        pltpu.sync_copy(zeros_hbm, acc.at[pl.ds(s * RPT, RPT)])
        pltpu.sync_copy(dstq_hbm.at[s, pl.ds(c * NCHD, NCHD)], dst_v)
        pltpu.sync_copy(ones_hbm, ones_v)
        plsc.subcore_barrier()

        @pl.loop(0, NCHD)
        def _(j):
            pltpu.sync_copy(ones_v, acc.at[dst_v.at[j]], add=True)

        plsc.subcore_barrier()
        pltpu.sync_copy(acc.at[pl.ds(s * RPT, RPT)],
                        out_hbm.at[c, pl.ds(s * RPT, RPT)])

    return deg_kernel


def _make_agg_kernel(dh, final=False):
    nbuf = 3
    out_ty = (jax.ShapeDtypeStruct((NN, NC * dh), jnp.float32) if final
              else jax.ShapeDtypeStruct((NC, NP, dh), jnp.float32))

    @functools.partial(
        pl.kernel,
        mesh=_MESH,
        compiler_params=_SC_PARAMS,
        out_type=out_ty,
        scratch_types=(
            [pltpu.VMEM((NCH2, CH), jnp.int32)]
            + [pltpu.VMEM((CH,), jnp.int32) for _ in range(nbuf)]
            + [pltpu.VMEM((CH, dh), jnp.float32) for _ in range(nbuf)]
            + [pltpu.VMEM_SHARED((NP, dh), jnp.float32)] * 2
            + [pltpu.SemaphoreType.DMA] * (2 * nbuf)
        ),
    )
    def agg_kernel(table2_hbm, srcq_hbm, dstq_hbm, init_hbm, out_hbm,
                   *refs):
        src_v = refs[0]
        dst_b = refs[1:1 + nbuf]
        rows_b = refs[1 + nbuf:1 + 2 * nbuf]
        table_s, acc = refs[1 + 2 * nbuf], refs[2 + 2 * nbuf]
        sem_r = refs[3 + 2 * nbuf:3 + 3 * nbuf]
        sem_d = refs[3 + 3 * nbuf:3 + 4 * nbuf]
        c = lax.axis_index("c")
        s = lax.axis_index("s")
        if final:
            pltpu.sync_copy(init_hbm.at[c], acc.at[pl.ds(s * RPT, RPT)])
        else:
            pltpu.sync_copy(init_hbm, acc.at[pl.ds(s * RPT, RPT)])
        pltpu.sync_copy(table2_hbm.at[c, pl.ds(s * RPT, RPT)],
                        table_s.at[pl.ds(s * RPT, RPT)])
        pltpu.sync_copy(srcq_hbm.at[s], src_v)
        plsc.subcore_barrier()

        for k in range(nbuf):
            pltpu.async_copy(dstq_hbm.at[s, k], dst_b[k], sem_d[k])
            pltpu.async_copy(table_s.at[src_v.at[k]], rows_b[k], sem_r[k])

        @pl.loop(0, NCH2 - nbuf, step=nbuf)
        def _(j):
            for k in range(nbuf):
                pltpu.make_async_copy(dstq_hbm.at[s, j], dst_b[k],
                                      sem_d[k]).wait()
                pltpu.make_async_copy(table_s.at[src_v.at[j]], rows_b[k],
                                      sem_r[k]).wait()
                pltpu.sync_copy(rows_b[k], acc.at[dst_b[k]], add=True)
                pltpu.async_copy(dstq_hbm.at[s, j + nbuf + k], dst_b[k],
                                 sem_d[k])
                pltpu.async_copy(table_s.at[src_v.at[j + nbuf + k]],
                                 rows_b[k], sem_r[k])

        for k in range(nbuf):
            pltpu.make_async_copy(dstq_hbm.at[s, 0], dst_b[k],
                                  sem_d[k]).wait()
            pltpu.make_async_copy(table_s.at[src_v.at[0]], rows_b[k],
                                  sem_r[k]).wait()
            pltpu.sync_copy(rows_b[k], acc.at[dst_b[k]], add=True)

        plsc.subcore_barrier()
        if final:
            rpn = NN // NS
            pltpu.sync_copy(
                acc.at[pl.ds(s * rpn, rpn)],
                out_hbm.at[pl.ds(s * rpn, rpn), pl.ds(c * dh, dh)])
        else:
            pltpu.sync_copy(acc.at[pl.ds(s * RPT, RPT)],
                            out_hbm.at[c, pl.ds(s * RPT, RPT)])

    return agg_kernel


_BLK = 256
_GRID = NP // _BLK
FH = FI // 2
CH2 = CC // 2


def _dinv_block(degp):
    deg = degp[0, :, 0:1] + degp[1, :, 0:1] + 1.0
    return lax.rsqrt(deg)


def _scale_body(degp_ref, x_ref, xs2_ref):
    dinv = _dinv_block(degp_ref[...])
    xs2_ref[0] = x_ref[:, :FH] * dinv
    xs2_ref[1] = x_ref[:, FH:] * dinv


def _mm_body(s2_ref, degp_ref, xs2_ref, w1_ref, b1_ref, w2_ref, p2_ref):
    dinv = _dinv_block(degp_ref[...])
    y0 = dinv * (s2_ref[0] + xs2_ref[0])
    y1 = dinv * (s2_ref[1] + xs2_ref[1])
    h = (jnp.dot(y0, w1_ref[0], preferred_element_type=jnp.float32,
                 precision=lax.Precision.HIGHEST)
         + jnp.dot(y1, w1_ref[1], preferred_element_type=jnp.float32,
                   precision=lax.Precision.HIGHEST))
    h = jnp.maximum(h + b1_ref[...], 0.0)
    p = jnp.dot(h, w2_ref[...], preferred_element_type=jnp.float32,
                precision=lax.Precision.HIGHEST)
    p2_ref[0] = p[:, :CH2]
    p2_ref[1] = p[:, CH2:]


def kernel(x, edge_index, W1, b1, W2, b2):
    src = edge_index[0].astype(jnp.int32)
    dst = edge_index[1].astype(jnp.int32)
    npad = EP - EE
    srcq = jnp.concatenate([src, jnp.zeros((npad,), jnp.int32)])
    srcq = srcq.reshape(NS, NCH2, CH)
    dstq = jnp.concatenate([dst, jnp.full((npad,), NN, jnp.int32)])
    dstq = dstq.reshape(NS, NCH2, CH)
    x_pad = jnp.pad(x, ((0, NP - NN), (0, 0)))
    ones8 = jnp.ones((CH, 8), jnp.float32)
    z8 = jnp.zeros((RPT, 8), jnp.float32)
    zfh = jnp.zeros((RPT, FH), jnp.float32)
    w1s = W1.reshape(2, FH, HH)
    b1r = b1.reshape(1, HH)
    b2init = jnp.broadcast_to(b2.reshape(NC, 1, CH2), (NC, RPT, CH2))

    degp = _make_deg_kernel()(dstq, ones8, z8)

    xs2 = pl.pallas_call(
        _scale_body,
        grid=(_GRID,),
        in_specs=[
            pl.BlockSpec((2, _BLK, 8), lambda i: (0, i, 0)),
            pl.BlockSpec((_BLK, FI), lambda i: (i, 0)),
        ],
        out_specs=pl.BlockSpec((2, _BLK, FH), lambda i: (0, i, 0)),
        out_shape=jax.ShapeDtypeStruct((2, NP, FH), jnp.float32),
    )(degp, x_pad)

    s2 = _make_agg_kernel(FH)(xs2, srcq, dstq, zfh)

    p2 = pl.pallas_call(
        _mm_body,
        grid=(_GRID,),
        in_specs=[
            pl.BlockSpec((2, _BLK, FH), lambda i: (0, i, 0)),
            pl.BlockSpec((2, _BLK, 8), lambda i: (0, i, 0)),
            pl.BlockSpec((2, _BLK, FH), lambda i: (0, i, 0)),
            pl.BlockSpec((2, FH, HH), lambda i: (0, 0, 0)),
            pl.BlockSpec((1, HH), lambda i: (0, 0)),
            pl.BlockSpec((HH, CC), lambda i: (0, 0)),
        ],
        out_specs=pl.BlockSpec((2, _BLK, CH2), lambda i: (0, i, 0)),
        out_shape=jax.ShapeDtypeStruct((2, NP, CH2), jnp.float32),
    )(s2, degp, xs2, w1s, b1r, W2)

    return _make_agg_kernel(CH2, final=True)(p2, srcq, dstq, b2init)


# speedup vs baseline: 23.2102x; 1.0022x over previous
import functools

import jax
import jax.numpy as jnp
from jax import lax
from jax.experimental import pallas as pl
from jax.experimental.pallas import tpu as pltpu
from jax.experimental.pallas import tpu_sc as plsc

NN = 10000
EE = 320000
FI = 128
HH = 256
CC = 64

NC = 2
NS = 16
CH = 128
NCH2 = 162
NCHD = NCH2 // NC
EP = NS * NCH2 * CH
NP = 10240
RPT = NP // NS

_MESH = plsc.VectorSubcoreMesh(core_axis_name="c", subcore_axis_name="s")
_SC_PARAMS = pltpu.CompilerParams(use_tc_tiling_on_sc=False)


FH = FI // 2
CH2 = CC // 2
RCH = 128
NRQ = RPT // RCH


def _rsqrt16(d):
    i = lax.bitcast_convert_type(d, jnp.int32)
    i = jnp.int32(0x5F3759DF) - lax.shift_right_logical(i, 1)
    y = lax.bitcast_convert_type(i, jnp.float32)
    for _ in range(3):
        y = y * (1.5 - 0.5 * d * y * y)
    return y


def _make_l1_kernel():
    @functools.partial(
        pl.kernel,
        mesh=_MESH,
        compiler_params=_SC_PARAMS,
        out_type=jax.ShapeDtypeStruct((NC, NP, FH), jnp.float32),
        scratch_types=(
            [pltpu.VMEM((CH,), jnp.int32) for _ in range(6)]
            + [pltpu.VMEM((CH,), jnp.int32) for _ in range(3)]
            + [pltpu.VMEM((CH, FH), jnp.float32) for _ in range(3)]
            + [pltpu.VMEM((CH, 16), jnp.float32)]
            + [pltpu.VMEM((RCH, 16), jnp.float32)]
            + [pltpu.VMEM_SHARED((NP, 16), jnp.float32)]
            + [pltpu.VMEM_SHARED((NP, FH), jnp.float32)]
            + [pltpu.VMEM_SHARED((NP, FH), jnp.float32)]
            + [pltpu.SemaphoreType.DMA] * 12
        ),
    )
    def l1_kernel(x_hbm, srcq_hbm, dstq_hbm, ones_hbm, z16_hbm, out_hbm,
                  *refs):
        sidx = refs[0:6]
        didx = refs[6:9]
        rows = refs[9:12]
        ones_v, degc = refs[12], refs[13]
        degacc, table_s, acc = refs[14], refs[15], refs[16]
        sem_s = refs[17:23]
        sem_d = refs[23:26]
        sem_r = refs[26:29]
        c = lax.axis_index("c")
        s = lax.axis_index("s")

        pltpu.sync_copy(z16_hbm, degacc.at[pl.ds(s * RPT, RPT)])
        pltpu.sync_copy(ones_hbm, ones_v)
        plsc.subcore_barrier()
        for k in range(3):
            pltpu.async_copy(dstq_hbm.at[s, k], didx[k], sem_d[k])

        @pl.loop(0, NCH2 - 3, step=3)
        def _(j):
            for k in range(3):
                pltpu.make_async_copy(dstq_hbm.at[s, j], didx[k],
                                      sem_d[k]).wait()
                pltpu.sync_copy(ones_v, degacc.at[didx[k]], add=True)
                pltpu.async_copy(dstq_hbm.at[s, j + 3 + k], didx[k],
                                 sem_d[k])

        for k in range(3):
            pltpu.make_async_copy(dstq_hbm.at[s, 0], didx[k],
                                  sem_d[k]).wait()
            pltpu.sync_copy(ones_v, degacc.at[didx[k]], add=True)
        plsc.subcore_barrier()

        for q in range(NRQ):
            r0 = s * RPT + q * RCH
            pltpu.sync_copy(degacc.at[pl.ds(r0, RCH)], degc)
            pltpu.sync_copy(x_hbm.at[pl.ds(r0, RCH), pl.ds(c * FH, FH)],
                            rows[0])

            @pl.loop(0, RCH)
            def _(r):
                y = _rsqrt16(degc[r, :] + 1.0)
                for c2 in range(FH // 16):
                    sl = pl.ds(c2 * 16, 16)
                    rows[0][r, sl] = rows[0][r, sl] * y

            pltpu.sync_copy(rows[0], table_s.at[pl.ds(r0, RCH)])
            pltpu.sync_copy(rows[0], acc.at[pl.ds(r0, RCH)])
        plsc.subcore_barrier()

        for k in range(3):
            pltpu.async_copy(srcq_hbm.at[s, k], sidx[k], sem_s[k])
            pltpu.async_copy(srcq_hbm.at[s, 3 + k], sidx[3 + k],
                             sem_s[3 + k])
            pltpu.async_copy(dstq_hbm.at[s, k], didx[k], sem_d[k])
        for k in range(3):
            pltpu.make_async_copy(srcq_hbm.at[s, 0], sidx[k],
                                  sem_s[k]).wait()
            pltpu.async_copy(table_s.at[sidx[k]], rows[k], sem_r[k])

        def _halfstep(j, k, pa, pb, g_next, s_next):
            pltpu.make_async_copy(dstq_hbm.at[s, 0], didx[k],
                                  sem_d[k]).wait()
            pltpu.make_async_copy(table_s.at[sidx[pa]], rows[k],
                                  sem_r[k]).wait()
            pltpu.sync_copy(rows[k], acc.at[didx[k]], add=True)
            pltpu.async_copy(dstq_hbm.at[s, g_next + k], didx[k], sem_d[k])
            pltpu.make_async_copy(srcq_hbm.at[s, 0], sidx[pb],
                                  sem_s[pb]).wait()
            pltpu.async_copy(table_s.at[sidx[pb]], rows[k], sem_r[k])
            pltpu.async_copy(srcq_hbm.at[s, s_next + k], sidx[pa],
                             sem_s[pa])

        @pl.loop(0, NCH2 - 6, step=6)
        def _(j):
            for k in range(3):
                _halfstep(j, k, k, 3 + k, j + 3, j + 6)
            for k in range(3):
                _halfstep(j + 3, k, 3 + k, k, j + 6, j + 9)

        for k in range(3):
            pltpu.make_async_copy(dstq_hbm.at[s, 0], didx[k],
                                  sem_d[k]).wait()
            pltpu.make_async_copy(table_s.at[sidx[k]], rows[k],
                                  sem_r[k]).wait()
            pltpu.sync_copy(rows[k], acc.at[didx[k]], add=True)
            pltpu.async_copy(dstq_hbm.at[s, NCH2 - 3 + k], didx[k],
                             sem_d[k])
            pltpu.make_async_copy(srcq_hbm.at[s, 0], sidx[3 + k],
                                  sem_s[3 + k]).wait()
            pltpu.async_copy(table_s.at[sidx[3 + k]], rows[k], sem_r[k])
        for k in range(3):
            pltpu.make_async_copy(dstq_hbm.at[s, 0], didx[k],
                                  sem_d[k]).wait()
            pltpu.make_async_copy(table_s.at[sidx[3 + k]], rows[k],
                                  sem_r[k]).wait()
            pltpu.sync_copy(rows[k], acc.at[didx[k]], add=True)
        plsc.subcore_barrier()

        for q in range(NRQ):
            r0 = s * RPT + q * RCH
            pltpu.sync_copy(degacc.at[pl.ds(r0, RCH)], degc)
            pltpu.sync_copy(acc.at[pl.ds(r0, RCH)], rows[0])

            @pl.loop(0, RCH)
            def _(r):
                y = _rsqrt16(degc[r, :] + 1.0)
                for c2 in range(FH // 16):
                    sl = pl.ds(c2 * 16, 16)
                    rows[0][r, sl] = rows[0][r, sl] * y

            pltpu.sync_copy(rows[0], out_hbm.at[c, pl.ds(r0, RCH)])

    return l1_kernel


def _make_agg_kernel(dh, final=False):
    nbuf = 3
    out_ty = (jax.ShapeDtypeStruct((NN, NC * dh), jnp.float32) if final
              else jax.ShapeDtypeStruct((NC, NP, dh), jnp.float32))

    @functools.partial(
        pl.kernel,
        mesh=_MESH,
        compiler_params=_SC_PARAMS,
        out_type=out_ty,
        scratch_types=(
            [pltpu.VMEM((NCH2, CH), jnp.int32)]
            + [pltpu.VMEM((CH,), jnp.int32) for _ in range(nbuf)]
            + [pltpu.VMEM((CH, dh), jnp.float32) for _ in range(nbuf)]
            + [pltpu.VMEM_SHARED((NP, dh), jnp.float32)] * 2
            + [pltpu.SemaphoreType.DMA] * (2 * nbuf)
        ),
    )
    def agg_kernel(table2_hbm, srcq_hbm, dstq_hbm, init_hbm, out_hbm,
                   *refs):
        src_v = refs[0]
        dst_b = refs[1:1 + nbuf]
        rows_b = refs[1 + nbuf:1 + 2 * nbuf]
        table_s, acc = refs[1 + 2 * nbuf], refs[2 + 2 * nbuf]
        sem_r = refs[3 + 2 * nbuf:3 + 3 * nbuf]
        sem_d = refs[3 + 3 * nbuf:3 + 4 * nbuf]
        c = lax.axis_index("c")
        s = lax.axis_index("s")
        if final:
            pltpu.sync_copy(init_hbm.at[c], acc.at[pl.ds(s * RPT, RPT)])
        else:
            pltpu.sync_copy(init_hbm, acc.at[pl.ds(s * RPT, RPT)])
        pltpu.sync_copy(table2_hbm.at[c, pl.ds(s * RPT, RPT)],
                        table_s.at[pl.ds(s * RPT, RPT)])
        pltpu.sync_copy(srcq_hbm.at[s], src_v)
        plsc.subcore_barrier()

        for k in range(nbuf):
            pltpu.async_copy(dstq_hbm.at[s, k], dst_b[k], sem_d[k])
            pltpu.async_copy(table_s.at[src_v.at[k]], rows_b[k], sem_r[k])

        @pl.loop(0, NCH2 - nbuf, step=nbuf)
        def _(j):
            for k in range(nbuf):
                pltpu.make_async_copy(dstq_hbm.at[s, j], dst_b[k],
                                      sem_d[k]).wait()
                pltpu.make_async_copy(table_s.at[src_v.at[j]], rows_b[k],
                                      sem_r[k]).wait()
                pltpu.sync_copy(rows_b[k], acc.at[dst_b[k]], add=True)
                pltpu.async_copy(dstq_hbm.at[s, j + nbuf + k], dst_b[k],
                                 sem_d[k])
                pltpu.async_copy(table_s.at[src_v.at[j + nbuf + k]],
                                 rows_b[k], sem_r[k])

        for k in range(nbuf):
            pltpu.make_async_copy(dstq_hbm.at[s, 0], dst_b[k],
                                  sem_d[k]).wait()
            pltpu.make_async_copy(table_s.at[src_v.at[0]], rows_b[k],
                                  sem_r[k]).wait()
            pltpu.sync_copy(rows_b[k], acc.at[dst_b[k]], add=True)

        plsc.subcore_barrier()
        if final:
            rpn = NN // NS
            pltpu.sync_copy(
                acc.at[pl.ds(s * rpn, rpn)],
                out_hbm.at[pl.ds(s * rpn, rpn), pl.ds(c * dh, dh)])
        else:
            pltpu.sync_copy(acc.at[pl.ds(s * RPT, RPT)],
                            out_hbm.at[c, pl.ds(s * RPT, RPT)])

    return agg_kernel


_BLK = 256
_GRID = NP // _BLK


def _mm_body(y2_ref, w1_ref, b1_ref, w2_ref, p2_ref):
    h = (jnp.dot(y2_ref[0], w1_ref[0], preferred_element_type=jnp.float32,
                 precision=lax.Precision.HIGHEST)
         + jnp.dot(y2_ref[1], w1_ref[1], preferred_element_type=jnp.float32,
                   precision=lax.Precision.HIGHEST))
    h = jnp.maximum(h + b1_ref[...], 0.0)
    p = jnp.dot(h, w2_ref[...], preferred_element_type=jnp.float32,
                precision=lax.Precision.HIGHEST)
    p2_ref[0] = p[:, :CH2]
    p2_ref[1] = p[:, CH2:]


def kernel(x, edge_index, W1, b1, W2, b2):
    src = edge_index[0].astype(jnp.int32)
    dst = edge_index[1].astype(jnp.int32)
    npad = EP - EE
    srcq = jnp.concatenate([src, jnp.zeros((npad,), jnp.int32)])
    srcq = srcq.reshape(NS, NCH2, CH)
    dstq = jnp.concatenate([dst, jnp.full((npad,), NN, jnp.int32)])
    dstq = dstq.reshape(NS, NCH2, CH)
    x_pad = jnp.pad(x, ((0, NP - NN), (0, 0)))
    ones16 = jnp.ones((CH, 16), jnp.float32)
    z16 = jnp.zeros((RPT, 16), jnp.float32)
    w1s = W1.reshape(2, FH, HH)
    b1r = b1.reshape(1, HH)
    b2init = jnp.broadcast_to(b2.reshape(NC, 1, CH2), (NC, RPT, CH2))

    y2 = _make_l1_kernel()(x_pad, srcq, dstq, ones16, z16)

    p2 = pl.pallas_call(
        _mm_body,
        grid=(_GRID,),
        in_specs=[
            pl.BlockSpec((2, _BLK, FH), lambda i: (0, i, 0)),
            pl.BlockSpec((2, FH, HH), lambda i: (0, 0, 0)),
            pl.BlockSpec((1, HH), lambda i: (0, 0)),
            pl.BlockSpec((HH, CC), lambda i: (0, 0)),
        ],
        out_specs=pl.BlockSpec((2, _BLK, CH2), lambda i: (0, i, 0)),
        out_shape=jax.ShapeDtypeStruct((2, NP, CH2), jnp.float32),
    )(y2, w1s, b1r, W2)

    return _make_agg_kernel(CH2, final=True)(p2, srcq, dstq, b2init)


# speedup vs baseline: 25.7086x; 1.1076x over previous
import functools

import jax
import jax.numpy as jnp
from jax import lax
from jax.experimental import pallas as pl
from jax.experimental.pallas import tpu as pltpu
from jax.experimental.pallas import tpu_sc as plsc

NN = 10000
EE = 320000
FI = 128
HH = 256
CC = 64

NC = 2
NS = 16
CH = 128
NCH2 = 162
NCHD = NCH2 // NC
EP = NS * NCH2 * CH
NP = 10240
RPT = NP // NS

_MESH = plsc.VectorSubcoreMesh(core_axis_name="c", subcore_axis_name="s")
_SC_PARAMS = pltpu.CompilerParams(use_tc_tiling_on_sc=False)


FH = FI // 2
CH2 = CC // 2
RCH = 128
NRQ = RPT // RCH


def _rsqrt16(d):
    i = lax.bitcast_convert_type(d, jnp.int32)
    i = jnp.int32(0x5F3759DF) - lax.shift_right_logical(i, 1)
    y = lax.bitcast_convert_type(i, jnp.float32)
    for _ in range(3):
        y = y * (1.5 - 0.5 * d * y * y)
    return y


def _make_l1_kernel():
    @functools.partial(
        pl.kernel,
        mesh=_MESH,
        compiler_params=_SC_PARAMS,
        out_type=jax.ShapeDtypeStruct((NC, NP, FH), jnp.float32),
        scratch_types=(
            [pltpu.VMEM((CH,), jnp.int32) for _ in range(6)]
            + [pltpu.VMEM((CH,), jnp.int32) for _ in range(3)]
            + [pltpu.VMEM((CH, FH), jnp.float32) for _ in range(3)]
            + [pltpu.VMEM((CH, 16), jnp.float32)]
            + [pltpu.VMEM((RCH, 16), jnp.float32)]
            + [pltpu.VMEM_SHARED((NP, 16), jnp.float32)]
            + [pltpu.VMEM_SHARED((NP, FH), jnp.float32)]
            + [pltpu.VMEM_SHARED((NP, FH), jnp.float32)]
            + [pltpu.SemaphoreType.DMA] * 12
        ),
    )
    def l1_kernel(x_hbm, srcq_hbm, dstq_hbm, ones_hbm, z16_hbm, out_hbm,
                  *refs):
        sidx = refs[0:6]
        didx = refs[6:9]
        rows = refs[9:12]
        ones_v, degc = refs[12], refs[13]
        degacc, table_s, acc = refs[14], refs[15], refs[16]
        sem_s = refs[17:23]
        sem_d = refs[23:26]
        sem_r = refs[26:29]
        c = lax.axis_index("c")
        s = lax.axis_index("s")

        pltpu.sync_copy(z16_hbm, degacc.at[pl.ds(s * RPT, RPT)])
        pltpu.sync_copy(ones_hbm, ones_v)
        plsc.subcore_barrier()
        for k in range(3):
            pltpu.async_copy(dstq_hbm.at[s, k], didx[k], sem_d[k])

        @pl.loop(0, NCH2 - 3, step=3)
        def _(j):
            for k in range(3):
                pltpu.make_async_copy(dstq_hbm.at[s, j], didx[k],
                                      sem_d[k]).wait()
                pltpu.sync_copy(ones_v, degacc.at[didx[k]], add=True)
                pltpu.async_copy(dstq_hbm.at[s, j + 3 + k], didx[k],
                                 sem_d[k])

        for k in range(3):
            pltpu.make_async_copy(dstq_hbm.at[s, 0], didx[k],
                                  sem_d[k]).wait()
            pltpu.sync_copy(ones_v, degacc.at[didx[k]], add=True)
        plsc.subcore_barrier()

        for q in range(NRQ):
            r0 = s * RPT + q * RCH
            pltpu.sync_copy(degacc.at[pl.ds(r0, RCH)], degc)
            pltpu.sync_copy(x_hbm.at[pl.ds(r0, RCH), pl.ds(c * FH, FH)],
                            rows[0])

            @pl.loop(0, RCH)
            def _(r):
                y = _rsqrt16(degc[r, :] + 1.0)
                for c2 in range(FH // 16):
                    sl = pl.ds(c2 * 16, 16)
                    rows[0][r, sl] = rows[0][r, sl] * y

            pltpu.sync_copy(rows[0], table_s.at[pl.ds(r0, RCH)])
            pltpu.sync_copy(rows[0], acc.at[pl.ds(r0, RCH)])
        plsc.subcore_barrier()

        for k in range(3):
            pltpu.async_copy(srcq_hbm.at[s, k], sidx[k], sem_s[k])
            pltpu.async_copy(srcq_hbm.at[s, 3 + k], sidx[3 + k],
                             sem_s[3 + k])
            pltpu.async_copy(dstq_hbm.at[s, k], didx[k], sem_d[k])
        for k in range(3):
            pltpu.make_async_copy(srcq_hbm.at[s, 0], sidx[k],
                                  sem_s[k]).wait()
            pltpu.async_copy(table_s.at[sidx[k]], rows[k], sem_r[k])

        def _halfstep(j, k, pa, pb, g_next, s_next):
            pltpu.make_async_copy(dstq_hbm.at[s, 0], didx[k],
                                  sem_d[k]).wait()
            pltpu.make_async_copy(table_s.at[sidx[pa]], rows[k],
                                  sem_r[k]).wait()
            pltpu.sync_copy(rows[k], acc.at[didx[k]], add=True)
            pltpu.async_copy(dstq_hbm.at[s, g_next + k], didx[k], sem_d[k])
            pltpu.make_async_copy(srcq_hbm.at[s, 0], sidx[pb],
                                  sem_s[pb]).wait()
            pltpu.async_copy(table_s.at[sidx[pb]], rows[k], sem_r[k])
            pltpu.async_copy(srcq_hbm.at[s, s_next + k], sidx[pa],
                             sem_s[pa])

        @pl.loop(0, NCH2 - 6, step=6)
        def _(j):
            for k in range(3):
                _halfstep(j, k, k, 3 + k, j + 3, j + 6)
            for k in range(3):
                _halfstep(j + 3, k, 3 + k, k, j + 6, j + 9)

        for k in range(3):
            pltpu.make_async_copy(dstq_hbm.at[s, 0], didx[k],
                                  sem_d[k]).wait()
            pltpu.make_async_copy(table_s.at[sidx[k]], rows[k],
                                  sem_r[k]).wait()
            pltpu.sync_copy(rows[k], acc.at[didx[k]], add=True)
            pltpu.async_copy(dstq_hbm.at[s, NCH2 - 3 + k], didx[k],
                             sem_d[k])
            pltpu.make_async_copy(srcq_hbm.at[s, 0], sidx[3 + k],
                                  sem_s[3 + k]).wait()
            pltpu.async_copy(table_s.at[sidx[3 + k]], rows[k], sem_r[k])
        for k in range(3):
            pltpu.make_async_copy(dstq_hbm.at[s, 0], didx[k],
                                  sem_d[k]).wait()
            pltpu.make_async_copy(table_s.at[sidx[3 + k]], rows[k],
                                  sem_r[k]).wait()
            pltpu.sync_copy(rows[k], acc.at[didx[k]], add=True)
        plsc.subcore_barrier()

        for q in range(NRQ):
            r0 = s * RPT + q * RCH
            pltpu.sync_copy(degacc.at[pl.ds(r0, RCH)], degc)
            pltpu.sync_copy(acc.at[pl.ds(r0, RCH)], rows[0])

            @pl.loop(0, RCH)
            def _(r):
                y = _rsqrt16(degc[r, :] + 1.0)
                for c2 in range(FH // 16):
                    sl = pl.ds(c2 * 16, 16)
                    rows[0][r, sl] = rows[0][r, sl] * y

            pltpu.sync_copy(rows[0], out_hbm.at[c, pl.ds(r0, RCH)])

    return l1_kernel


def _make_agg_kernel(dh, final=False):
    nbuf = 3
    out_ty = (jax.ShapeDtypeStruct((NN, NC * dh), jnp.float32) if final
              else jax.ShapeDtypeStruct((NC, NP, dh), jnp.float32))

    @functools.partial(
        pl.kernel,
        mesh=_MESH,
        compiler_params=_SC_PARAMS,
        out_type=out_ty,
        scratch_types=(
            [pltpu.VMEM((NCH2, CH), jnp.int32)]
            + [pltpu.VMEM((CH,), jnp.int32) for _ in range(nbuf)]
            + [pltpu.VMEM((CH, dh), jnp.float32) for _ in range(nbuf)]
            + [pltpu.VMEM_SHARED((NP, dh), jnp.float32)] * 2
            + [pltpu.SemaphoreType.DMA] * (2 * nbuf)
        ),
    )
    def agg_kernel(table2_hbm, srcq_hbm, dstq_hbm, init_hbm, out_hbm,
                   *refs):
        src_v = refs[0]
        dst_b = refs[1:1 + nbuf]
        rows_b = refs[1 + nbuf:1 + 2 * nbuf]
        table_s, acc = refs[1 + 2 * nbuf], refs[2 + 2 * nbuf]
        sem_r = refs[3 + 2 * nbuf:3 + 3 * nbuf]
        sem_d = refs[3 + 3 * nbuf:3 + 4 * nbuf]
        c = lax.axis_index("c")
        s = lax.axis_index("s")
        if final:
            pltpu.sync_copy(init_hbm.at[c], acc.at[pl.ds(s * RPT, RPT)])
        else:
            pltpu.sync_copy(init_hbm, acc.at[pl.ds(s * RPT, RPT)])
        pltpu.sync_copy(table2_hbm.at[c, pl.ds(s * RPT, RPT)],
                        table_s.at[pl.ds(s * RPT, RPT)])
        pltpu.sync_copy(srcq_hbm.at[s], src_v)
        plsc.subcore_barrier()

        for k in range(nbuf):
            pltpu.async_copy(dstq_hbm.at[s, k], dst_b[k], sem_d[k])
            pltpu.async_copy(table_s.at[src_v.at[k]], rows_b[k], sem_r[k])

        @pl.loop(0, NCH2 - nbuf, step=nbuf)
        def _(j):
            for k in range(nbuf):
                pltpu.make_async_copy(dstq_hbm.at[s, j], dst_b[k],
                                      sem_d[k]).wait()
                pltpu.make_async_copy(table_s.at[src_v.at[j]], rows_b[k],
                                      sem_r[k]).wait()
                pltpu.sync_copy(rows_b[k], acc.at[dst_b[k]], add=True)
                pltpu.async_copy(dstq_hbm.at[s, j + nbuf + k], dst_b[k],
                                 sem_d[k])
                pltpu.async_copy(table_s.at[src_v.at[j + nbuf + k]],
                                 rows_b[k], sem_r[k])

        for k in range(nbuf):
            pltpu.make_async_copy(dstq_hbm.at[s, 0], dst_b[k],
                                  sem_d[k]).wait()
            pltpu.make_async_copy(table_s.at[src_v.at[0]], rows_b[k],
                                  sem_r[k]).wait()
            pltpu.sync_copy(rows_b[k], acc.at[dst_b[k]], add=True)

        plsc.subcore_barrier()
        if final:
            rpn = NN // NS
            pltpu.sync_copy(
                acc.at[pl.ds(s * rpn, rpn)],
                out_hbm.at[pl.ds(s * rpn, rpn), pl.ds(c * dh, dh)])
        else:
            pltpu.sync_copy(acc.at[pl.ds(s * RPT, RPT)],
                            out_hbm.at[c, pl.ds(s * RPT, RPT)])

    return agg_kernel


_BLK = 1024
_GRID = NP // _BLK


def _mm_body(y2_ref, w1_ref, b1_ref, w2_ref, p2_ref):
    h = (jnp.dot(y2_ref[0], w1_ref[0], preferred_element_type=jnp.float32)
         + jnp.dot(y2_ref[1], w1_ref[1], preferred_element_type=jnp.float32))
    h = jnp.maximum(h + b1_ref[...], 0.0)
    p = jnp.dot(h, w2_ref[...], preferred_element_type=jnp.float32)
    p2_ref[0] = p[:, :CH2]
    p2_ref[1] = p[:, CH2:]


def kernel(x, edge_index, W1, b1, W2, b2):
    src = edge_index[0].astype(jnp.int32)
    dst = edge_index[1].astype(jnp.int32)
    npad = EP - EE
    srcq = jnp.concatenate([src, jnp.zeros((npad,), jnp.int32)])
    srcq = srcq.reshape(NS, NCH2, CH)
    dstq = jnp.concatenate([dst, jnp.full((npad,), NN, jnp.int32)])
    dstq = dstq.reshape(NS, NCH2, CH)
    x_pad = jnp.pad(x, ((0, NP - NN), (0, 0)))
    ones16 = jnp.ones((CH, 16), jnp.float32)
    z16 = jnp.zeros((RPT, 16), jnp.float32)
    w1s = W1.reshape(2, FH, HH)
    b1r = b1.reshape(1, HH)
    b2init = jnp.broadcast_to(b2.reshape(NC, 1, CH2), (NC, RPT, CH2))

    y2 = _make_l1_kernel()(x_pad, srcq, dstq, ones16, z16)

    p2 = pl.pallas_call(
        _mm_body,
        grid=(_GRID,),
        in_specs=[
            pl.BlockSpec((2, _BLK, FH), lambda i: (0, i, 0)),
            pl.BlockSpec((2, FH, HH), lambda i: (0, 0, 0)),
            pl.BlockSpec((1, HH), lambda i: (0, 0)),
            pl.BlockSpec((HH, CC), lambda i: (0, 0)),
        ],
        out_specs=pl.BlockSpec((2, _BLK, CH2), lambda i: (0, i, 0)),
        out_shape=jax.ShapeDtypeStruct((2, NP, CH2), jnp.float32),
    )(y2, w1s, b1r, W2)

    return _make_agg_kernel(CH2, final=True)(p2, srcq, dstq, b2init)
